# baseline (speedup 1.0000x reference)
import functools

import jax
import jax.numpy as jnp
from jax import lax
from jax.experimental import pallas as pl
from jax.experimental.pallas import tpu as pltpu
from jax.experimental.pallas import tpu_sc as plsc

N_CORES = 2
N_SUBCORES = 16
N_WORKERS = N_CORES * N_SUBCORES
LANE = 16
SK = 16
UPB = 16
EPB = SK * UPB
HALF = 5120
NA = 5248


@functools.lru_cache(maxsize=None)
def _make_spmm(n, d, epw):
    mesh = plsc.VectorSubcoreMesh(core_axis_name="c", subcore_axis_name="s")
    rpt = n // N_SUBCORES
    rpa = NA // N_SUBCORES
    assert n % N_SUBCORES == 0 and rpt % 8 == 0
    nb = epw // EPB
    assert epw % (2 * EPB) == 0

    idx_t = [
        pltpu.VMEM((EPB,), jnp.int32),
        pltpu.VMEM((EPB,), jnp.int32),
        pltpu.VMEM((EPB,), jnp.float32),
    ]

    @functools.partial(
        pl.kernel,
        out_type=jax.ShapeDtypeStruct((N_CORES, 2, NA, d), jnp.float32),
        mesh=mesh,
        scratch_types=(
            idx_t + idx_t
            + [pltpu.VMEM((SK, d), jnp.float32)]
            + [pltpu.VMEM((SK, d), jnp.float32)] * 2
            + [pltpu.VMEM((SK,), jnp.int32)] * 2
            + [pltpu.SemaphoreType.DMA]
            + [pltpu.SemaphoreType.DMA] * 2
            + [pltpu.SemaphoreType.DMA] * 2
            + [pltpu.VMEM_SHARED((n, d), jnp.float32)]
            + [pltpu.VMEM_SHARED((NA, d), jnp.float32)]
        ),
    )
    def spmm(x_hbm, src_hbm, dst_hbm, val_hbm, out_hbm, *scr):
        srcb = (scr[0], scr[3])
        dstb = (scr[1], scr[4])
        valb = (scr[2], scr[5])
        gbuf = scr[6]
        sbuf = scr[7:9]
        dstq = scr[9:11]
        gsem = scr[11]
        ssem = scr[12:14]
        isem = scr[14:16]
        xs = scr[16]
        acc = scr[17]

        cid = lax.axis_index("c")
        sid = lax.axis_index("s")
        wid = cid * N_SUBCORES + sid
        ebase = wid * epw
        xbase = sid * rpt
        abase = sid * rpa

        def zero_acc():
            def zb(r, carry):
                for j in range(d // LANE):
                    sbuf[0][r, pl.ds(j * LANE, LANE)] = jnp.zeros(
                        (LANE,), jnp.float32)
                return carry
            lax.fori_loop(0, SK, zb, 0)
            for i in range(rpa // 8):
                pltpu.sync_copy(sbuf[0].at[pl.ds(0, 8)],
                                acc.at[pl.ds(abase + i * 8, 8)])

        zero_acc()
        pltpu.sync_copy(x_hbm.at[pl.ds(xbase, rpt)],
                        xs.at[pl.ds(xbase, rpt)])
        plsc.subcore_barrier()

        def issue_gather(h, u):
            return pltpu.async_copy(
                xs.at[srcb[h].at[pl.ds(u * SK, SK)]], gbuf, gsem)

        def wait_gather(h):
            pltpu.make_async_copy(
                xs.at[srcb[h].at[pl.ds(0, SK)]], gbuf, gsem).wait()

        def wait_scatter(k):
            pltpu.make_async_copy(
                sbuf[k], acc.at[dstq[k]], ssem[k]).wait()

        def scale(h, u, k):
            vv = valb[h][pl.ds(u * SK, SK)]
            for r in range(SK):
                v = vv[r]
                for j in range(d // LANE):
                    sl = pl.ds(j * LANE, LANE)
                    sbuf[k][r, sl] = gbuf[r, sl] * v

        def load_block(b, h):
            off = ebase + b * EPB
            pltpu.async_copy(src_hbm.at[pl.ds(off, EPB)], srcb[h], isem[h])
            pltpu.async_copy(dst_hbm.at[pl.ds(off, EPB)], dstb[h], isem[h])
            pltpu.async_copy(val_hbm.at[pl.ds(off, EPB)], valb[h], isem[h])

        def wait_block(h):
            pltpu.make_async_copy(
                src_hbm.at[pl.ds(0, EPB)], srcb[h], isem[h]).wait()
            pltpu.make_async_copy(
                dst_hbm.at[pl.ds(0, EPB)], dstb[h], isem[h]).wait()
            pltpu.make_async_copy(
                val_hbm.at[pl.ds(0, EPB)], valb[h], isem[h]).wait()

        def process_block(h, ph):
            issue_gather(h, 0)

            def body(uu, carry):
                for k in range(2):
                    u = 2 * uu + k
                    wait_gather(h)

                    @pl.when(uu > 0)
                    def _():
                        wait_scatter(k)

                    scale(h, u, k)

                    @pl.when(u + 1 < UPB)
                    def _():
                        issue_gather(h, u + 1)

                    dv = dstb[h][pl.ds(u * SK, SK)]
                    if ph == 0:
                        dq = jnp.where(dv < HALF, dv, HALF)
                    else:
                        dq = jnp.where(dv >= HALF, dv - HALF, HALF)
                    dstq[k][pl.ds(0, SK)] = dq
                    pltpu.async_copy(
                        sbuf[k], acc.at[dstq[k]], ssem[k], add=True)
                return carry

            lax.fori_loop(0, UPB // 2, body, 0)
            wait_scatter(0)
            wait_scatter(1)

        for ph in range(2):
            load_block(0, 0)

            def pairbody(t, carry, ph=ph):
                b0 = 2 * t
                load_block(b0 + 1, 1)
                wait_block(0)
                process_block(0, ph)

                @pl.when(b0 + 2 < nb)
                def _():
                    load_block(b0 + 2, 0)

                wait_block(1)
                process_block(1, ph)
                return carry

            lax.fori_loop(0, nb // 2, pairbody, 0)
            plsc.subcore_barrier()

            pltpu.sync_copy(acc.at[pl.ds(abase, rpa)],
                            out_hbm.at[cid, ph, pl.ds(abase, rpa)])
            if ph == 0:
                zero_acc()
                plsc.subcore_barrier()

    return spmm


def _combine_matmul(p, w, bias, n):
    d = p.shape[3]
    d_out = w.shape[1]
    bm = 80
    assert n % bm == 0 and HALF % bm == 0
    nlow = HALF // bm

    def body(p_ref, w_ref, b_ref, o_ref):
        s = p_ref[0, 0] + p_ref[1, 0]
        o_ref[...] = jnp.dot(
            s, w_ref[...], preferred_element_type=jnp.float32) + b_ref[...]

    def pmap_(i):
        return (0, jnp.where(i < nlow, 0, 1),
                jnp.where(i < nlow, i, i - nlow), 0)

    return pl.pallas_call(
        body,
        grid=(n // bm,),
        in_specs=[
            pl.BlockSpec((2, 1, bm, d), pmap_),
            pl.BlockSpec((d, d_out), lambda i: (0, 0)),
            pl.BlockSpec((1, d_out), lambda i: (0, 0)),
        ],
        out_specs=pl.BlockSpec((bm, d_out), lambda i: (i, 0)),
        out_shape=jax.ShapeDtypeStruct((n, d_out), jnp.float32),
    )(p, w, bias.reshape(1, d_out))


def kernel(x, edge_index, edge_vals, W, bias):
    n, d = x.shape
    e = edge_vals.shape[0]
    src = edge_index[0].astype(jnp.int32)
    dst = edge_index[1].astype(jnp.int32)
    vals = edge_vals.astype(jnp.float32)

    quantum = N_WORKERS * 2 * EPB
    e_pad = -(-e // quantum) * quantum
    epw = e_pad // N_WORKERS
    if e_pad > e:
        pad = e_pad - e
        src = jnp.concatenate([src, jnp.zeros((pad,), jnp.int32)])
        dst = jnp.concatenate([dst, jnp.zeros((pad,), jnp.int32)])
        vals = jnp.concatenate([vals, jnp.zeros((pad,), jnp.float32)])

    n_pad = -(-n // (8 * N_SUBCORES)) * (8 * N_SUBCORES)
    if n_pad > n:
        x = jnp.pad(x, ((0, n_pad - n), (0, 0)))

    partials = _make_spmm(n_pad, d, epw)(x, src, dst, vals)
    return _combine_matmul(partials, W, bias, n)
